# baseline (speedup 1.0000x reference)
import functools

import jax
import jax.numpy as jnp
from jax import lax
from jax.experimental import pallas as pl
from jax.experimental.pallas import tpu as pltpu
from jax.experimental.pallas import tpu_sc as plsc

_DIM = 64
_NC = 2
_NS = 16
_NW = _NC * _NS
_CHUNK = 512


def _sc_gather(table, idx, width=_DIM, tc_tiling=False):
    nw, nch, c = idx.shape
    npw = nch * c
    mesh = plsc.VectorSubcoreMesh(core_axis_name="c", subcore_axis_name="s")

    @functools.partial(
        pl.kernel,
        mesh=mesh,
        out_type=jax.ShapeDtypeStruct((nw * nch, c, width), jnp.float32),
        compiler_params=pltpu.CompilerParams(use_tc_tiling_on_sc=tc_tiling),
        scratch_types=[
            pltpu.VMEM((npw,), jnp.int32),
            pltpu.VMEM((1, c, width), jnp.float32),
            pltpu.SemaphoreType.DMA,
        ],
    )
    def gk(table_hbm, idx_hbm, out_hbm, idx_v, buf, sem):
        wid = lax.axis_index("s") * _NC + lax.axis_index("c")
        pltpu.sync_copy(idx_hbm.at[wid], idx_v)

        def body(j, carry):
            pltpu.async_copy(table_hbm.at[idx_v.at[pl.ds(j * c, c)]],
                             buf.at[0], sem).wait()
            pltpu.sync_copy(buf, out_hbm.at[pl.ds(wid * nch + j, 1)])
            return carry

        lax.fori_loop(0, nch, body, 0)

    return gk(table, idx.reshape(nw, npw)).reshape(nw * npw, width)


def _tc_body(ent_ref, par_ref, rel_ref, sgn_ref, out_ref):
    i = pl.program_id(0)

    def ent(k):
        full = ent_ref[k]
        par = par_ref[:, k : k + 1]
        return jnp.where(par > 0, full[:, _DIM:], full[:, :_DIM])

    eh = ent(0)
    et = ent(1)
    ehn = ent(2)
    ern = ent(3)
    etn = ent(4)
    er = rel_ref[0]

    def sq_norm(x):
        return jnp.sqrt(jnp.sum(x * x, axis=1, keepdims=True))

    base = er - et
    tmps = []
    scores = []
    for j in range(10):
        tmp = ent(5 + j) - rel_ref[1 + j]
        scores.append(-sq_norm(base - tmp))
        tmps.append(tmp)
    m = scores[0]
    for j in range(1, 10):
        m = jnp.maximum(m, scores[j])
    es = [jnp.exp(s - m) for s in scores]
    ssum = es[0]
    for j in range(1, 10):
        ssum = ssum + es[j]
    agg = es[0] * tmps[0]
    for j in range(1, 10):
        agg = agg + es[j] * tmps[j]
    agg = agg / ssum
    g_n_pos = -sq_norm(agg - eh)
    g_n_neg = -sq_norm(agg - ehn)

    eps = []
    pscores = []
    for p in range(10):
        ep = None
        for l in range(3):
            col = sgn_ref[:, 3 * p + l : 3 * p + l + 1]
            contrib = jnp.where(col > 0, rel_ref[11 + 3 * p + l], 0.0)
            ep = contrib if ep is None else ep + contrib
        pscores.append(-sq_norm(eh + ep - et))
        eps.append(ep)
    pm = pscores[0]
    for p in range(1, 10):
        pm = jnp.maximum(pm, pscores[p])
    pes = [jnp.exp(s - pm) for s in pscores]
    psum = pes[0]
    for p in range(1, 10):
        psum = psum + pes[p]
    pagg = pes[0] * eps[0]
    for p in range(1, 10):
        pagg = pagg + pes[p] * eps[p]
    pagg = pagg / psum
    g_p_pos = -sq_norm(pagg - er)
    g_p_neg = -sq_norm(pagg - ern)

    g_t_pos = -sq_norm(eh + er - et)
    g_t_neg = -sq_norm(ehn + ern - etn)

    def hinge(pos, neg):
        return jnp.sum(jnp.maximum(neg - pos + 1.0, 0.0))

    lsum = (hinge(g_n_pos, g_n_neg) + hinge(g_p_pos, g_p_neg)
            + hinge(g_t_pos, g_t_neg))

    @pl.when(i == 0)
    def _():
        out_ref[...] = jnp.zeros_like(out_ref)

    out_ref[...] += jnp.reshape(lsum, (1, 1))


def _tc_loss(g_ent, parity, g_rel, path_ids, block_rows):
    b = g_ent.shape[1]
    nblk = b // block_rows
    return pl.pallas_call(
        _tc_body,
        grid=(nblk,),
        in_specs=[
            pl.BlockSpec((15, block_rows, 2 * _DIM), lambda i: (0, i, 0)),
            pl.BlockSpec((block_rows, 15), lambda i: (i, 0)),
            pl.BlockSpec((41, block_rows, _DIM), lambda i: (0, i, 0)),
            pl.BlockSpec((block_rows, 30), lambda i: (i, 0)),
        ],
        out_specs=pl.BlockSpec((1, 1), lambda i: (0, 0)),
        out_shape=jax.ShapeDtypeStruct((1, 1), jnp.float32),
    )(g_ent, parity, g_rel, path_ids)


def kernel(entity_table, relation_table, h_batch, r_batch, t_batch,
           h_neg_batch, r_neg_batch, t_neg_batch, neighbor_context,
           path_context):
    b = h_batch.shape[0]
    i32 = jnp.int32

    nc_t = neighbor_context[:, :, 1].astype(i32).T
    nc_r = neighbor_context[:, :, 0].astype(i32).T
    path_ids2d = path_context.astype(i32).reshape(b, 30)
    path_t = jnp.transpose(path_context.astype(i32), (1, 2, 0)).reshape(30, b)

    ent_idx = jnp.concatenate(
        [jnp.stack([h_batch.astype(i32), t_batch.astype(i32),
                    h_neg_batch.astype(i32), r_neg_batch.astype(i32),
                    t_neg_batch.astype(i32)]),
         nc_t], axis=0)
    rel_idx = jnp.concatenate(
        [r_batch.astype(i32)[None, :], nc_r, path_t], axis=0)

    nch_e = (15 * b) // (_NW * _CHUNK)
    nch_r = (41 * b) // (_NW * _CHUNK)
    ent_pair = ent_idx // 2
    parity = (ent_idx & 1).T
    g_ent = _sc_gather(entity_table.reshape(-1, 2 * _DIM),
                       ent_pair.reshape(_NW, nch_e, _CHUNK),
                       width=2 * _DIM, tc_tiling=True)
    g_rel = _sc_gather(relation_table,
                       rel_idx.reshape(_NW, nch_r, _CHUNK))

    loss = _tc_loss(g_ent.reshape(15, b, 2 * _DIM), parity,
                    g_rel.reshape(41, b, _DIM),
                    path_ids2d, 512)
    return loss[0, 0]


# speedup vs baseline: 1.1461x; 1.1461x over previous
import functools

import jax
import jax.numpy as jnp
from jax import lax
from jax.experimental import pallas as pl
from jax.experimental.pallas import tpu as pltpu
from jax.experimental.pallas import tpu_sc as plsc

_DIM = 64
_NC = 2
_NS = 16
_NW = _NC * _NS
_CHUNK = 128


def _sc_gather(table, idx):
    nw, nch, c = idx.shape
    npw = nch * c
    mesh = plsc.VectorSubcoreMesh(core_axis_name="c", subcore_axis_name="s")

    @functools.partial(
        pl.kernel,
        mesh=mesh,
        out_type=jax.ShapeDtypeStruct((nw * npw, _DIM), jnp.float32),
        compiler_params=pltpu.CompilerParams(use_tc_tiling_on_sc=False),
        scratch_types=[
            pltpu.VMEM((nch, c), jnp.int32),
            pltpu.VMEM((c, _DIM), jnp.float32),
            pltpu.SemaphoreType.DMA,
        ],
    )
    def gk(table_hbm, idx_hbm, out_hbm, idx_v, buf, sem):
        wid = lax.axis_index("s") * _NC + lax.axis_index("c")
        pltpu.sync_copy(idx_hbm.at[wid], idx_v)

        def body(g, carry):
            pltpu.async_copy(table_hbm.at[idx_v.at[g]], buf, sem).wait()
            pltpu.sync_copy(buf, out_hbm.at[pl.ds(wid * npw + g * c, c)])
            return carry

        lax.fori_loop(0, nch, body, 0)

    return gk(table, idx)


def _tc_body(ent_ref, rel_ref, sgn_ref, out_ref):
    i = pl.program_id(0)
    eh = ent_ref[0]
    et = ent_ref[1]
    ehn = ent_ref[2]
    ern = ent_ref[3]
    etn = ent_ref[4]
    er = rel_ref[0]

    def sq_norm(x):
        return jnp.sqrt(jnp.sum(x * x, axis=1, keepdims=True))

    base = er - et
    tmps = []
    scores = []
    for j in range(10):
        tmp = ent_ref[5 + j] - rel_ref[1 + j]
        scores.append(-sq_norm(base - tmp))
        tmps.append(tmp)
    m = scores[0]
    for j in range(1, 10):
        m = jnp.maximum(m, scores[j])
    es = [jnp.exp(s - m) for s in scores]
    ssum = es[0]
    for j in range(1, 10):
        ssum = ssum + es[j]
    agg = es[0] * tmps[0]
    for j in range(1, 10):
        agg = agg + es[j] * tmps[j]
    agg = agg / ssum
    g_n_pos = -sq_norm(agg - eh)
    g_n_neg = -sq_norm(agg - ehn)

    eps = []
    pscores = []
    for p in range(10):
        ep = None
        for l in range(3):
            col = sgn_ref[:, 3 * p + l : 3 * p + l + 1]
            contrib = jnp.where(col > 0, rel_ref[11 + 3 * p + l], 0.0)
            ep = contrib if ep is None else ep + contrib
        pscores.append(-sq_norm(eh + ep - et))
        eps.append(ep)
    pm = pscores[0]
    for p in range(1, 10):
        pm = jnp.maximum(pm, pscores[p])
    pes = [jnp.exp(s - pm) for s in pscores]
    psum = pes[0]
    for p in range(1, 10):
        psum = psum + pes[p]
    pagg = pes[0] * eps[0]
    for p in range(1, 10):
        pagg = pagg + pes[p] * eps[p]
    pagg = pagg / psum
    g_p_pos = -sq_norm(pagg - er)
    g_p_neg = -sq_norm(pagg - ern)

    g_t_pos = -sq_norm(eh + er - et)
    g_t_neg = -sq_norm(ehn + ern - etn)

    def hinge(pos, neg):
        return jnp.sum(jnp.maximum(neg - pos + 1.0, 0.0))

    lsum = (hinge(g_n_pos, g_n_neg) + hinge(g_p_pos, g_p_neg)
            + hinge(g_t_pos, g_t_neg))

    @pl.when(i == 0)
    def _():
        out_ref[...] = jnp.zeros_like(out_ref)

    out_ref[...] += jnp.reshape(lsum, (1, 1))


def _tc_loss(g_ent, g_rel, path_ids, block_rows):
    b = g_ent.shape[1]
    nblk = b // block_rows
    return pl.pallas_call(
        _tc_body,
        grid=(nblk,),
        in_specs=[
            pl.BlockSpec((15, block_rows, _DIM), lambda i: (0, i, 0)),
            pl.BlockSpec((41, block_rows, _DIM), lambda i: (0, i, 0)),
            pl.BlockSpec((block_rows, 30), lambda i: (i, 0)),
        ],
        out_specs=pl.BlockSpec((1, 1), lambda i: (0, 0)),
        out_shape=jax.ShapeDtypeStruct((1, 1), jnp.float32),
    )(g_ent, g_rel, path_ids)


def _half(entity_table, relation_table, h, r, t, hn, rn, tn, nbr, path):
    b = h.shape[0]
    i32 = jnp.int32

    nc_t = nbr[:, :, 1].astype(i32).T
    nc_r = nbr[:, :, 0].astype(i32).T
    path_ids2d = path.astype(i32).reshape(b, 30)
    path_t = jnp.transpose(path.astype(i32), (1, 2, 0)).reshape(30, b)

    ent_idx = jnp.concatenate(
        [jnp.stack([h.astype(i32), t.astype(i32), hn.astype(i32),
                    rn.astype(i32), tn.astype(i32)]),
         nc_t], axis=0)
    rel_idx = jnp.concatenate(
        [r.astype(i32)[None, :], nc_r, path_t], axis=0)

    nch_e = (15 * b) // (_NW * _CHUNK)
    nch_r = (41 * b) // (_NW * _CHUNK)
    g_ent = _sc_gather(entity_table, ent_idx.reshape(_NW, nch_e, _CHUNK))
    g_rel = _sc_gather(relation_table, rel_idx.reshape(_NW, nch_r, _CHUNK))

    return _tc_loss(g_ent.reshape(15, b, _DIM),
                    g_rel.reshape(41, b, _DIM),
                    path_ids2d, 512)


def kernel(entity_table, relation_table, h_batch, r_batch, t_batch,
           h_neg_batch, r_neg_batch, t_neg_batch, neighbor_context,
           path_context):
    b = h_batch.shape[0]
    h2 = b // 2

    def part(lo, hi):
        return _half(entity_table, relation_table,
                     h_batch[lo:hi], r_batch[lo:hi], t_batch[lo:hi],
                     h_neg_batch[lo:hi], r_neg_batch[lo:hi],
                     t_neg_batch[lo:hi], neighbor_context[lo:hi],
                     path_context[lo:hi])

    loss = part(0, h2) + part(h2, b)
    return loss[0, 0]


# speedup vs baseline: 1.3607x; 1.1872x over previous
import functools

import jax
import jax.numpy as jnp
from jax import lax
from jax.experimental import pallas as pl
from jax.experimental.pallas import tpu as pltpu
from jax.experimental.pallas import tpu_sc as plsc

_DIM = 64
_NC = 2
_NS = 16
_NW = _NC * _NS
_CHUNK = 128


def _sc_gather(table, idx):
    nw, nch, c = idx.shape
    npw = nch * c
    mesh = plsc.VectorSubcoreMesh(core_axis_name="c", subcore_axis_name="s")

    @functools.partial(
        pl.kernel,
        mesh=mesh,
        out_type=jax.ShapeDtypeStruct((nw * npw, _DIM), jnp.float32),
        compiler_params=pltpu.CompilerParams(use_tc_tiling_on_sc=False),
        scratch_types=[
            pltpu.VMEM((nch, c), jnp.int32),
            pltpu.VMEM((c, _DIM), jnp.float32),
            pltpu.SemaphoreType.DMA,
        ],
    )
    def gk(table_hbm, idx_hbm, out_hbm, idx_v, buf, sem):
        wid = lax.axis_index("s") * _NC + lax.axis_index("c")
        pltpu.sync_copy(idx_hbm.at[wid], idx_v)

        def body(g, carry):
            pltpu.async_copy(table_hbm.at[idx_v.at[g]], buf, sem).wait()
            pltpu.sync_copy(buf, out_hbm.at[pl.ds(wid * npw + g * c, c)])
            return carry

        lax.fori_loop(0, nch, body, 0)

    return gk(table, idx)


def _tc_body(ent_ref, rel_ref, sgn_ref, out_ref):
    i = pl.program_id(0)

    def sq_norm(x):
        return jnp.sqrt(jnp.sum(x * x, axis=1, keepdims=True))

    lsum = None
    for hh in range(2):
        lo = hh * _DIM

        def ent(k):
            return ent_ref[k][:, lo : lo + _DIM]

        def rel(k):
            return rel_ref[k][:, lo : lo + _DIM]

        eh = ent(0)
        et = ent(1)
        ehn = ent(2)
        ern = ent(3)
        etn = ent(4)
        er = rel(0)

        base = er - et
        tmps = []
        scores = []
        for j in range(10):
            tmp = ent(5 + j) - rel(1 + j)
            scores.append(-sq_norm(base - tmp))
            tmps.append(tmp)
        m = scores[0]
        for j in range(1, 10):
            m = jnp.maximum(m, scores[j])
        es = [jnp.exp(s - m) for s in scores]
        ssum = es[0]
        for j in range(1, 10):
            ssum = ssum + es[j]
        agg = es[0] * tmps[0]
        for j in range(1, 10):
            agg = agg + es[j] * tmps[j]
        agg = agg / ssum
        g_n_pos = -sq_norm(agg - eh)
        g_n_neg = -sq_norm(agg - ehn)

        eps = []
        pscores = []
        for p in range(10):
            ep = None
            for l in range(3):
                ci = 30 * hh + 3 * p + l
                col = sgn_ref[:, ci : ci + 1]
                contrib = jnp.where(col > 0, rel(11 + 3 * p + l), 0.0)
                ep = contrib if ep is None else ep + contrib
            pscores.append(-sq_norm(eh + ep - et))
            eps.append(ep)
        pm = pscores[0]
        for p in range(1, 10):
            pm = jnp.maximum(pm, pscores[p])
        pes = [jnp.exp(s - pm) for s in pscores]
        psum = pes[0]
        for p in range(1, 10):
            psum = psum + pes[p]
        pagg = pes[0] * eps[0]
        for p in range(1, 10):
            pagg = pagg + pes[p] * eps[p]
        pagg = pagg / psum
        g_p_pos = -sq_norm(pagg - er)
        g_p_neg = -sq_norm(pagg - ern)

        g_t_pos = -sq_norm(eh + er - et)
        g_t_neg = -sq_norm(ehn + ern - etn)

        def hinge(pos, neg):
            return jnp.sum(jnp.maximum(neg - pos + 1.0, 0.0))

        part = (hinge(g_n_pos, g_n_neg) + hinge(g_p_pos, g_p_neg)
                + hinge(g_t_pos, g_t_neg))
        lsum = part if lsum is None else lsum + part

    @pl.when(i == 0)
    def _():
        out_ref[...] = jnp.zeros_like(out_ref)

    out_ref[...] += jnp.reshape(lsum, (1, 1))


def _tc_loss(g_ent, g_rel, path_ids, block_rows):
    b2 = g_ent.shape[1]
    nblk = b2 // block_rows
    return pl.pallas_call(
        _tc_body,
        grid=(nblk,),
        in_specs=[
            pl.BlockSpec((15, block_rows, 2 * _DIM), lambda i: (0, i, 0)),
            pl.BlockSpec((41, block_rows, 2 * _DIM), lambda i: (0, i, 0)),
            pl.BlockSpec((block_rows, 60), lambda i: (i, 0)),
        ],
        out_specs=pl.BlockSpec((1, 1), lambda i: (0, 0)),
        out_shape=jax.ShapeDtypeStruct((1, 1), jnp.float32),
    )(g_ent, g_rel, path_ids)


def _half(entity_table, relation_table, h, r, t, hn, rn, tn, nbr, path):
    b = h.shape[0]
    i32 = jnp.int32

    nc_t = nbr[:, :, 1].astype(i32).T
    nc_r = nbr[:, :, 0].astype(i32).T
    path_ids2d = path.astype(i32).reshape(b, 30)
    path_t = jnp.transpose(path.astype(i32), (1, 2, 0)).reshape(30, b)

    ent_idx = jnp.concatenate(
        [jnp.stack([h.astype(i32), t.astype(i32), hn.astype(i32),
                    rn.astype(i32), tn.astype(i32)]),
         nc_t], axis=0)
    rel_idx = jnp.concatenate(
        [r.astype(i32)[None, :], nc_r, path_t], axis=0)

    nch_e = (15 * b) // (_NW * _CHUNK)
    nch_r = (41 * b) // (_NW * _CHUNK)
    g_ent = _sc_gather(entity_table, ent_idx.reshape(_NW, nch_e, _CHUNK))
    g_rel = _sc_gather(relation_table, rel_idx.reshape(_NW, nch_r, _CHUNK))

    return _tc_loss(g_ent.reshape(15, b // 2, 2 * _DIM),
                    g_rel.reshape(41, b // 2, 2 * _DIM),
                    path_ids2d.reshape(b // 2, 60), 512)


def kernel(entity_table, relation_table, h_batch, r_batch, t_batch,
           h_neg_batch, r_neg_batch, t_neg_batch, neighbor_context,
           path_context):
    loss = _half(entity_table, relation_table, h_batch, r_batch, t_batch,
                 h_neg_batch, r_neg_batch, t_neg_batch, neighbor_context,
                 path_context)
    return loss[0, 0]
